# baseline (speedup 1.0000x reference)
import jax
import jax.numpy as jnp
from jax.experimental import pallas as pl
from jax.experimental.pallas import tpu as pltpu


def _make_transpose_moments_kernel(c_in, n_hp, n_p, w_img, tn):
    k = c_in * w_img

    def body(x_ref, xt_ref, m_ref, s_ref):
        @pl.when(pl.program_id(1) == 0)
        def _init():
            m_ref[...] = jnp.zeros_like(m_ref)
            s_ref[...] = jnp.zeros_like(s_ref)

        xs = x_ref[...]
        m = m_ref[0]
        for j in range(n_hp):
            for p in range(n_p):
                slab = xs[:, j * n_p + p, :, :].reshape(k, tn)
                m += jax.lax.dot_general(
                    slab, slab, (((1,), (1,)), ((), ())),
                    preferred_element_type=jnp.float32)
                xt_ref[:, j, k * p:k * (p + 1)] = jnp.transpose(slab, (1, 0))
        m_ref[0] = m
        s_ref[0] += jnp.sum(xs, axis=(1, 2, 3)).reshape(c_in, 1)

    return body


def _make_deconv_kernel(c_out, n_p, k, v, nb, hp):
    def body(x_ref, w_ref, sh_ref, o_ref):
        xa = x_ref[...].reshape(nb * hp, n_p * k)
        sh = sh_ref[...]
        ys = []
        for p in range(n_p):
            y = jnp.dot(xa[:, k * p:k * (p + 1)], w_ref[...],
                        preferred_element_type=jnp.float32)
            ys.append(jnp.maximum(y + sh, 0.0))
        for c in range(c_out):
            ycat = jnp.concatenate(
                [ys[p][:, v * c:v * (c + 1)] for p in range(n_p)], axis=1)
            o_ref[:, c] = ycat.reshape(nb, hp, n_p * v)

    return body


def kernel(x, weight, bias, gamma, beta):
    eps = 1e-5
    N, C_in, H, W = x.shape
    _, C_out, kH, kW = weight.shape
    assert (kH, kW) == (2, 2), "specialized to kernel=2, stride=2, padding=0"
    del bias

    f32 = jnp.float32
    x = x.astype(f32)
    weight = weight.astype(f32)
    gamma = gamma.astype(f32)
    beta = beta.astype(f32)

    HW = H * W
    V = 4 * W
    K = C_in * W
    P = max(1, 128 // V)
    while H % P:
        P //= 2
    Hp = H // P

    xv = jnp.transpose(x, (1, 2, 3, 0))
    TN = 128 if N % 128 == 0 else N
    n_tn = N // TN
    THp = 8
    while Hp % THp:
        THp //= 2
    xt, m, s = pl.pallas_call(
        _make_transpose_moments_kernel(C_in, THp, P, W, TN),
        out_shape=(jax.ShapeDtypeStruct((N, Hp, P * K), f32),
                   jax.ShapeDtypeStruct((n_tn, K, K), f32),
                   jax.ShapeDtypeStruct((n_tn, C_in, 1), f32)),
        grid=(n_tn, Hp // THp),
        in_specs=[pl.BlockSpec((C_in, THp * P, W, TN),
                               lambda tn, th: (0, th, 0, tn))],
        out_specs=(pl.BlockSpec((TN, THp, P * K), lambda tn, th: (tn, th, 0)),
                   pl.BlockSpec((1, K, K), lambda tn, th: (tn, 0, 0)),
                   pl.BlockSpec((1, C_in, 1), lambda tn, th: (tn, 0, 0))),
        compiler_params=pltpu.CompilerParams(
            dimension_semantics=("parallel", "arbitrary")),
    )(xv)
    M = jnp.sum(m, axis=0).reshape(C_in, W, C_in, W)
    G = jnp.einsum('awbw->ab', M)
    sx = jnp.sum(s, axis=0)[:, 0]

    w2 = jnp.transpose(weight, (1, 2, 3, 0)).reshape(C_out * 4, C_in)
    sum_r = w2 @ sx
    sq_r = jnp.sum((w2 @ G) * w2, axis=1)
    count = jnp.asarray(N * 4 * HW, f32)
    mean = sum_r.reshape(C_out, 4).sum(axis=1) / count
    var = jnp.maximum(sq_r.reshape(C_out, 4).sum(axis=1) / count - mean * mean,
                      0.0)
    scale = gamma * jax.lax.rsqrt(var + eps)
    shift = beta - scale * mean

    u = (jnp.arange(2)[:, None, None] * (2 * W)
         + jnp.arange(W)[None, :, None] * 2
         + jnp.arange(2)[None, None, :])
    onehot = (u[..., None] == jnp.arange(V)).astype(f32)
    wall = jnp.einsum('ioab,awbu,o->iwou', weight, onehot,
                      scale).reshape(K, C_out * V)
    sh_l = jnp.repeat(shift, V).reshape(1, C_out * V)

    NB = 64
    while N % NB:
        NB //= 2
    xT = xt
    out = pl.pallas_call(
        _make_deconv_kernel(C_out, P, K, V, NB, Hp),
        out_shape=jax.ShapeDtypeStruct((N, C_out, Hp, P * V), f32),
        grid=(N // NB,),
        in_specs=[
            pl.BlockSpec((NB, Hp, P * K), lambda n: (n, 0, 0)),
            pl.BlockSpec((K, C_out * V), lambda n: (0, 0)),
            pl.BlockSpec((1, C_out * V), lambda n: (0, 0)),
        ],
        out_specs=pl.BlockSpec((NB, C_out, Hp, P * V),
                               lambda n: (n, 0, 0, 0)),
        compiler_params=pltpu.CompilerParams(
            dimension_semantics=("parallel",)),
    )(xT, wall, sh_l)

    return out.reshape(N, C_out, 2 * H, 2 * W)


# speedup vs baseline: 1.0162x; 1.0162x over previous
import jax
import jax.numpy as jnp
from jax.experimental import pallas as pl
from jax.experimental.pallas import tpu as pltpu


def _make_transpose_moments_kernel(c_in, n_hp, n_p, w_img, tn):
    k = c_in * w_img

    def body(x_ref, xt_ref, m_ref, s_ref):
        @pl.when(pl.program_id(1) == 0)
        def _init():
            m_ref[...] = jnp.zeros_like(m_ref)
            s_ref[...] = jnp.zeros_like(s_ref)

        xs = x_ref[...]
        m = m_ref[0]
        for j in range(n_hp):
            for p in range(n_p):
                slab = xs[:, j * n_p + p, :, :].reshape(k, tn)
                m += jax.lax.dot_general(
                    slab, slab, (((1,), (1,)), ((), ())),
                    preferred_element_type=jnp.float32)
                xt_ref[:, j, k * p:k * (p + 1)] = jnp.transpose(slab, (1, 0))
        m_ref[0] = m
        s_ref[0] += jnp.sum(xs, axis=(1, 2, 3)).reshape(c_in, 1)

    return body


def _make_deconv_kernel(c_out, n_p, k, v, nb, hp):
    def body(x_ref, w_ref, sh_ref, o_ref):
        xa = x_ref[...].reshape(nb * hp, n_p * k)
        sh = sh_ref[...]
        ys = []
        for p in range(n_p):
            y = jnp.dot(xa[:, k * p:k * (p + 1)], w_ref[...],
                        preferred_element_type=jnp.float32)
            ys.append(jnp.maximum(y + sh, 0.0))
        for c in range(c_out):
            ycat = jnp.concatenate(
                [ys[p][:, v * c:v * (c + 1)] for p in range(n_p)], axis=1)
            o_ref[:, c] = ycat.reshape(nb, hp, n_p * v)

    return body


def kernel(x, weight, bias, gamma, beta):
    eps = 1e-5
    N, C_in, H, W = x.shape
    _, C_out, kH, kW = weight.shape
    assert (kH, kW) == (2, 2), "specialized to kernel=2, stride=2, padding=0"
    del bias

    f32 = jnp.float32
    x = x.astype(f32)
    weight = weight.astype(f32)
    gamma = gamma.astype(f32)
    beta = beta.astype(f32)

    HW = H * W
    V = 4 * W
    K = C_in * W
    P = max(1, 128 // V)
    while H % P:
        P //= 2
    Hp = H // P

    xv = jnp.transpose(x, (1, 2, 3, 0))
    TN = 128 if N % 128 == 0 else N
    n_tn = N // TN
    THp = 16
    while Hp % THp:
        THp //= 2
    xt, m, s = pl.pallas_call(
        _make_transpose_moments_kernel(C_in, THp, P, W, TN),
        out_shape=(jax.ShapeDtypeStruct((N, Hp, P * K), f32),
                   jax.ShapeDtypeStruct((n_tn, K, K), f32),
                   jax.ShapeDtypeStruct((n_tn, C_in, 1), f32)),
        grid=(n_tn, Hp // THp),
        in_specs=[pl.BlockSpec((C_in, THp * P, W, TN),
                               lambda tn, th: (0, th, 0, tn))],
        out_specs=(pl.BlockSpec((TN, THp, P * K), lambda tn, th: (tn, th, 0)),
                   pl.BlockSpec((1, K, K), lambda tn, th: (tn, 0, 0)),
                   pl.BlockSpec((1, C_in, 1), lambda tn, th: (tn, 0, 0))),
        compiler_params=pltpu.CompilerParams(
            dimension_semantics=("parallel", "arbitrary")),
    )(xv)
    M = jnp.sum(m, axis=0).reshape(C_in, W, C_in, W)
    G = jnp.einsum('awbw->ab', M)
    sx = jnp.sum(s, axis=0)[:, 0]

    w2 = jnp.transpose(weight, (1, 2, 3, 0)).reshape(C_out * 4, C_in)
    sum_r = w2 @ sx
    sq_r = jnp.sum((w2 @ G) * w2, axis=1)
    count = jnp.asarray(N * 4 * HW, f32)
    mean = sum_r.reshape(C_out, 4).sum(axis=1) / count
    var = jnp.maximum(sq_r.reshape(C_out, 4).sum(axis=1) / count - mean * mean,
                      0.0)
    scale = gamma * jax.lax.rsqrt(var + eps)
    shift = beta - scale * mean

    u = (jnp.arange(2)[:, None, None] * (2 * W)
         + jnp.arange(W)[None, :, None] * 2
         + jnp.arange(2)[None, None, :])
    onehot = (u[..., None] == jnp.arange(V)).astype(f32)
    wall = jnp.einsum('ioab,awbu,o->iwou', weight, onehot,
                      scale).reshape(K, C_out * V)
    sh_l = jnp.repeat(shift, V).reshape(1, C_out * V)

    NB = 32
    while N % NB:
        NB //= 2
    xT = xt
    out = pl.pallas_call(
        _make_deconv_kernel(C_out, P, K, V, NB, Hp),
        out_shape=jax.ShapeDtypeStruct((N, C_out, Hp, P * V), f32),
        grid=(N // NB,),
        in_specs=[
            pl.BlockSpec((NB, Hp, P * K), lambda n: (n, 0, 0)),
            pl.BlockSpec((K, C_out * V), lambda n: (0, 0)),
            pl.BlockSpec((1, C_out * V), lambda n: (0, 0)),
        ],
        out_specs=pl.BlockSpec((NB, C_out, Hp, P * V),
                               lambda n: (n, 0, 0, 0)),
        compiler_params=pltpu.CompilerParams(
            dimension_semantics=("parallel",)),
    )(xT, wall, sh_l)

    return out.reshape(N, C_out, 2 * H, 2 * W)


# speedup vs baseline: 1.0169x; 1.0007x over previous
import jax
import jax.numpy as jnp
from jax.experimental import pallas as pl
from jax.experimental.pallas import tpu as pltpu


def _make_transpose_moments_kernel(c_in, n_hp, n_p, w_img, tn):
    k = c_in * w_img

    def body(x_ref, xt_ref, m_ref, s_ref):
        @pl.when(pl.program_id(1) == 0)
        def _init():
            m_ref[...] = jnp.zeros_like(m_ref)
            s_ref[...] = jnp.zeros_like(s_ref)

        xs = x_ref[...]
        m = m_ref[0]
        for j in range(n_hp):
            for p in range(n_p):
                slab = xs[:, j * n_p + p, :, :].reshape(k, tn)
                m += jax.lax.dot_general(
                    slab, slab, (((1,), (1,)), ((), ())),
                    preferred_element_type=jnp.float32)
                xt_ref[:, j, k * p:k * (p + 1)] = jnp.transpose(slab, (1, 0))
        m_ref[0] = m
        s_ref[0] += jnp.sum(xs, axis=(1, 2, 3)).reshape(c_in, 1)

    return body


def _make_deconv_kernel(c_out, n_p, k, v, nb, hp):
    def body(x_ref, w_ref, sh_ref, o_ref):
        xa = x_ref[...].reshape(nb * hp, n_p * k)
        sh = sh_ref[...]
        ys = []
        for p in range(n_p):
            y = jnp.dot(xa[:, k * p:k * (p + 1)], w_ref[...],
                        preferred_element_type=jnp.float32)
            ys.append(jnp.maximum(y + sh, 0.0))
        for c in range(c_out):
            ycat = jnp.concatenate(
                [ys[p][:, v * c:v * (c + 1)] for p in range(n_p)], axis=1)
            o_ref[:, c] = ycat.reshape(nb, hp, n_p * v)

    return body


def kernel(x, weight, bias, gamma, beta):
    eps = 1e-5
    N, C_in, H, W = x.shape
    _, C_out, kH, kW = weight.shape
    assert (kH, kW) == (2, 2), "specialized to kernel=2, stride=2, padding=0"
    del bias

    f32 = jnp.float32
    x = x.astype(f32)
    weight = weight.astype(f32)
    gamma = gamma.astype(f32)
    beta = beta.astype(f32)

    HW = H * W
    V = 4 * W
    K = C_in * W
    P = max(1, 128 // V)
    while H % P:
        P //= 2
    Hp = H // P

    xv = jnp.transpose(x, (1, 2, 3, 0))
    TN = 128 if N % 128 == 0 else N
    n_tn = N // TN
    THp = 32
    while Hp % THp:
        THp //= 2
    xt, m, s = pl.pallas_call(
        _make_transpose_moments_kernel(C_in, THp, P, W, TN),
        out_shape=(jax.ShapeDtypeStruct((N, Hp, P * K), f32),
                   jax.ShapeDtypeStruct((n_tn, K, K), f32),
                   jax.ShapeDtypeStruct((n_tn, C_in, 1), f32)),
        grid=(n_tn, Hp // THp),
        in_specs=[pl.BlockSpec((C_in, THp * P, W, TN),
                               lambda tn, th: (0, th, 0, tn))],
        out_specs=(pl.BlockSpec((TN, THp, P * K), lambda tn, th: (tn, th, 0)),
                   pl.BlockSpec((1, K, K), lambda tn, th: (tn, 0, 0)),
                   pl.BlockSpec((1, C_in, 1), lambda tn, th: (tn, 0, 0))),
        compiler_params=pltpu.CompilerParams(
            dimension_semantics=("parallel", "arbitrary")),
    )(xv)
    M = jnp.sum(m, axis=0).reshape(C_in, W, C_in, W)
    G = jnp.einsum('awbw->ab', M)
    sx = jnp.sum(s, axis=0)[:, 0]

    w2 = jnp.transpose(weight, (1, 2, 3, 0)).reshape(C_out * 4, C_in)
    sum_r = w2 @ sx
    sq_r = jnp.sum((w2 @ G) * w2, axis=1)
    count = jnp.asarray(N * 4 * HW, f32)
    mean = sum_r.reshape(C_out, 4).sum(axis=1) / count
    var = jnp.maximum(sq_r.reshape(C_out, 4).sum(axis=1) / count - mean * mean,
                      0.0)
    scale = gamma * jax.lax.rsqrt(var + eps)
    shift = beta - scale * mean

    u = (jnp.arange(2)[:, None, None] * (2 * W)
         + jnp.arange(W)[None, :, None] * 2
         + jnp.arange(2)[None, None, :])
    onehot = (u[..., None] == jnp.arange(V)).astype(f32)
    wall = jnp.einsum('ioab,awbu,o->iwou', weight, onehot,
                      scale).reshape(K, C_out * V)
    sh_l = jnp.repeat(shift, V).reshape(1, C_out * V)

    NB = 32
    while N % NB:
        NB //= 2
    xT = xt
    out = pl.pallas_call(
        _make_deconv_kernel(C_out, P, K, V, NB, Hp),
        out_shape=jax.ShapeDtypeStruct((N, C_out, Hp, P * V), f32),
        grid=(N // NB,),
        in_specs=[
            pl.BlockSpec((NB, Hp, P * K), lambda n: (n, 0, 0)),
            pl.BlockSpec((K, C_out * V), lambda n: (0, 0)),
            pl.BlockSpec((1, C_out * V), lambda n: (0, 0)),
        ],
        out_specs=pl.BlockSpec((NB, C_out, Hp, P * V),
                               lambda n: (n, 0, 0, 0)),
        compiler_params=pltpu.CompilerParams(
            dimension_semantics=("parallel",)),
    )(xT, wall, sh_l)

    return out.reshape(N, C_out, 2 * H, 2 * W)
